# baseline (speedup 1.0000x reference)
import functools

import jax
import jax.numpy as jnp
from jax import lax
from jax.experimental import pallas as pl
from jax.experimental.pallas import tpu as pltpu
from jax.experimental.pallas import tpu_sc as plsc

MAX_LEN = 200
EMBED = 64
NC, NS = 2, 16
NW = NC * NS
LANES = 16
NA, NB = 128, 80
NSEG = 216


@functools.lru_cache(maxsize=None)
def _make_kernel(n_batch: int):
    n_chunks = n_batch // NW
    assert n_chunks % 2 == 0
    mesh = plsc.VectorSubcoreMesh(core_axis_name="c", subcore_axis_name="s")

    ibuf = lambda n: pltpu.VMEM((n,), jnp.int32)
    fbuf = lambda n: pltpu.VMEM((n, EMBED), jnp.float32)

    @functools.partial(
        pl.kernel,
        mesh=mesh,
        compiler_params=pltpu.CompilerParams(use_tc_tiling_on_sc=False),
        out_type=jax.ShapeDtypeStruct((n_batch, MAX_LEN, EMBED), jnp.float32),
        scratch_types=[
            [ibuf(NA), ibuf(NA)],
            [ibuf(NB), ibuf(NB)],
            [ibuf(NSEG), ibuf(NSEG)],
            [ibuf(NA), ibuf(NA)],
            [ibuf(NB), ibuf(NB)],
            [fbuf(NA), fbuf(NA)],
            [fbuf(NB), fbuf(NB)],
            pltpu.VMEM_SHARED((4 * MAX_LEN, EMBED), jnp.float32),
            [pltpu.SemaphoreType.DMA, pltpu.SemaphoreType.DMA],
            [pltpu.SemaphoreType.DMA, pltpu.SemaphoreType.DMA],
            [pltpu.SemaphoreType.DMA, pltpu.SemaphoreType.DMA],
        ],
    )
    def k(seq_hbm, seg_hbm, tok_hbm, com2_hbm, out_hbm,
          idxA, idxB, segi, cidxA, cidxB, bufA, bufB, com2_sh,
          gsem, asem, osem):
        cid = lax.axis_index("c")
        sid = lax.axis_index("s")
        wid = sid * NC + cid
        w_b0 = wid * n_chunks
        iota = lax.broadcasted_iota(jnp.int32, (LANES,), 0)

        @pl.when(sid == 0)
        def _():
            pltpu.sync_copy(com2_hbm, com2_sh)
        plsc.subcore_barrier()

        def prefetch(c, p):
            base = (w_b0 + c) * MAX_LEN
            pltpu.sync_copy(seq_hbm.at[pl.ds(base, NA)], idxA[p])
            pltpu.sync_copy(seq_hbm.at[pl.ds(base + NA, NB)], idxB[p])
            pltpu.sync_copy(seg_hbm.at[pl.ds(base, NSEG)], segi[p])

        def gather_copies(p):
            return [
                pltpu.make_async_copy(tok_hbm.at[idxA[p]], bufA[p], gsem[p]),
                pltpu.make_async_copy(tok_hbm.at[idxB[p]], bufB[p], gsem[p]),
            ]

        def start_adds(p):
            return [
                pltpu.async_copy(com2_sh.at[cidxA[p]], bufA[p], asem[p],
                                 add=True),
                pltpu.async_copy(com2_sh.at[cidxB[p]], bufB[p], asem[p],
                                 add=True),
            ]

        def out_copies(c, p):
            bb = w_b0 + c
            return [
                pltpu.make_async_copy(bufA[p], out_hbm.at[bb, pl.ds(0, NA)],
                                      osem[p]),
                pltpu.make_async_copy(bufB[p].at[pl.ds(0, MAX_LEN - NA)],
                                      out_hbm.at[bb, pl.ds(NA, MAX_LEN - NA)],
                                      osem[p]),
            ]

        def halfstep(c, p, q):
            @pl.when(c > 0)
            def _():
                for cp in out_copies(c - 1, q):
                    cp.wait()

            @pl.when(c + 1 < n_chunks)
            def _():
                for cp in gather_copies(q):
                    cp.start()

            for cp in gather_copies(p):
                cp.wait()
            for g in range(NA // LANES):
                sl = pl.ds(g * LANES, LANES)
                cidxA[p][sl] = segi[p][sl] * (2 * MAX_LEN) + (
                    iota + (g * LANES))
            for g in range(NB // LANES):
                sl = pl.ds(g * LANES, LANES)
                cidxB[p][sl] = segi[p][pl.ds(NA + g * LANES, LANES)] * (
                    2 * MAX_LEN) + (iota + (NA + g * LANES))
            adds = start_adds(p)
            for cp in adds:
                cp.wait()
            for cp in out_copies(c, p):
                cp.start()

            @pl.when(c + 2 < n_chunks)
            def _():
                prefetch(c + 2, p)

        prefetch(0, 0)
        prefetch(1, 1)
        for cp in gather_copies(0):
            cp.start()

        def body(t, carry):
            halfstep(2 * t, 0, 1)
            halfstep(2 * t + 1, 1, 0)
            return carry

        lax.fori_loop(0, n_chunks // 2, body, 0)
        for cp in out_copies(n_chunks - 1, 1):
            cp.wait()

    return k


N_SLICES = 4


def kernel(seq, seg, token_emb, seg_emb, pos_emb):
    b, l = seq.shape
    seq_flat = jnp.pad(seq.reshape(-1).astype(jnp.int32), (0, LANES))
    seg_flat = jnp.pad(seg.reshape(-1).astype(jnp.int32), (0, LANES))
    pos2 = jnp.tile(pos_emb, (2, 1))
    com2 = jnp.concatenate([pos2 + seg_emb[0], pos2 + seg_emb[1]], axis=0)
    bs = b // N_SLICES
    rows = bs * l
    k = _make_kernel(bs)
    outs = [
        k(lax.dynamic_slice_in_dim(seq_flat, i * rows, rows + LANES),
          lax.dynamic_slice_in_dim(seg_flat, i * rows, rows + LANES),
          token_emb, com2)
        for i in range(N_SLICES)
    ]
    return jnp.concatenate(outs, axis=0)


# speedup vs baseline: 1.2633x; 1.2633x over previous
import functools

import jax
import jax.numpy as jnp
from jax import lax
from jax.experimental import pallas as pl
from jax.experimental.pallas import tpu as pltpu
from jax.experimental.pallas import tpu_sc as plsc

MAX_LEN = 200
EMBED = 64
NC, NS = 2, 16
NW = NC * NS
CHUNK = 128
LANES = 16


@functools.lru_cache(maxsize=None)
def _make_kernel(n_rows: int):
    rows_per_w = n_rows // NW
    n_chunks = rows_per_w // CHUNK
    assert rows_per_w % CHUNK == 0 and n_chunks % 2 == 0
    mesh = plsc.VectorSubcoreMesh(core_axis_name="c", subcore_axis_name="s")

    buf2 = lambda *shape: pltpu.VMEM(shape, jnp.int32)

    @functools.partial(
        pl.kernel,
        mesh=mesh,
        compiler_params=pltpu.CompilerParams(use_tc_tiling_on_sc=False),
        out_type=jax.ShapeDtypeStruct((n_rows, EMBED), jnp.float32),
        scratch_types=[
            [buf2(CHUNK), buf2(CHUNK)],
            [buf2(CHUNK), buf2(CHUNK)],
            pltpu.VMEM((CHUNK,), jnp.int32),
            pltpu.VMEM((CHUNK,), jnp.int32),
            [pltpu.VMEM((CHUNK, EMBED), jnp.float32),
             pltpu.VMEM((CHUNK, EMBED), jnp.float32)],
            pltpu.VMEM_SHARED((4 * MAX_LEN, EMBED), jnp.float32),
            [pltpu.SemaphoreType.DMA, pltpu.SemaphoreType.DMA],
            [pltpu.SemaphoreType.DMA, pltpu.SemaphoreType.DMA],
        ],
    )
    def k(seq_hbm, seg_hbm, tok_hbm, com2_hbm, out_hbm,
          idx_v, segi_v, cidx_v, ident_v, outbuf, com2_sh, gsem, osem):
        cid = lax.axis_index("c")
        sid = lax.axis_index("s")
        wid = sid * NC + cid
        w_base = wid * rows_per_w
        iota = lax.broadcasted_iota(jnp.int32, (LANES,), 0)
        for g in range(CHUNK // LANES):
            ident_v[pl.ds(g * LANES, LANES)] = iota + (g * LANES)

        @pl.when(sid == 0)
        def _():
            pltpu.sync_copy(com2_hbm, com2_sh)
        plsc.subcore_barrier()

        def prefetch(c, p):
            base = w_base + c * CHUNK
            pltpu.sync_copy(seq_hbm.at[pl.ds(base, CHUNK)], idx_v[p])
            pltpu.sync_copy(seg_hbm.at[pl.ds(base, CHUNK)], segi_v[p])

        def issue_gather(p):
            return pltpu.async_copy(tok_hbm.at[idx_v[p]], outbuf[p], gsem[p])

        def halfstep(c, p, q):
            base = w_base + c * CHUNK
            l0 = lax.rem(base, MAX_LEN)

            @pl.when(c > 0)
            def _():
                pltpu.make_async_copy(outbuf[q], out_hbm.at[pl.ds(0, CHUNK)],
                                      osem[q]).wait()

            @pl.when(c + 1 < n_chunks)
            def _():
                issue_gather(q)

            pltpu.make_async_copy(tok_hbm.at[idx_v[p]], outbuf[p],
                                  gsem[p]).wait()
            for g in range(CHUNK // LANES):
                sl = pl.ds(g * LANES, LANES)
                cidx_v[sl] = segi_v[p][sl] * (2 * MAX_LEN) + ident_v[sl] + l0
            pltpu.sync_copy(com2_sh.at[cidx_v], outbuf[p], add=True)
            pltpu.async_copy(outbuf[p], out_hbm.at[pl.ds(base, CHUNK)],
                             osem[p])

            @pl.when(c + 2 < n_chunks)
            def _():
                prefetch(c + 2, p)

        prefetch(0, 0)
        prefetch(1, 1)
        issue_gather(0)

        def body(t, carry):
            halfstep(2 * t, 0, 1)
            halfstep(2 * t + 1, 1, 0)
            return carry

        lax.fori_loop(0, n_chunks // 2, body, 0)
        pltpu.make_async_copy(outbuf[1], out_hbm.at[pl.ds(0, CHUNK)],
                              osem[1]).wait()

    return k


N_SLICES = 4


def kernel(seq, seg, token_emb, seg_emb, pos_emb):
    b, l = seq.shape
    seq_flat = seq.reshape(-1).astype(jnp.int32)
    seg_flat = seg.reshape(-1).astype(jnp.int32)
    pos2 = jnp.tile(pos_emb, (2, 1))
    com2 = jnp.concatenate([pos2 + seg_emb[0], pos2 + seg_emb[1]], axis=0)
    bs = b // N_SLICES
    rows = bs * l
    k = _make_kernel(rows)
    outs = [
        k(lax.dynamic_slice_in_dim(seq_flat, i * rows, rows),
          lax.dynamic_slice_in_dim(seg_flat, i * rows, rows),
          token_emb, com2).reshape(bs, l, EMBED)
        for i in range(N_SLICES)
    ]
    return jnp.concatenate(outs, axis=0)
